# baseline (speedup 1.0000x reference)
import jax
import jax.numpy as jnp
from jax import lax
from jax.experimental import pallas as pl
from jax.experimental.pallas import tpu as pltpu
from jax.experimental.pallas import tpu_sc as plsc

N = 10000
D = 128
E = 320000

NC = 2
NS = 16
NW = NC * NS
CHUNK = 128
NROWS_E = E // CHUNK
HROWS = 40
ROWS_A = 80
ROWS_B = 72
NTILES_A = 24
MIXROWS = 8
NACC = 10240
ROWS_PER_TILE = NACC // NS

_mesh = plsc.VectorSubcoreMesh(core_axis_name="c", subcore_axis_name="s")


def _spmm_body(x_hbm, rows_hbm, cols_hbm, vals_hbm,
               mrows_hbm, mcols_hbm, mvals_hbm, out_hbm,
               cols_v, rows_v, vals_v, buf0, buf1, acc, semg0, semg1):
    c = lax.axis_index("c")
    s = lax.axis_index("s")
    wid = s * NC + c

    with jax.named_scope("acc_zero"):
        def zero_body(e, carry):
            for q in range(D // 16):
                buf0[e, pl.ds(q * 16, 16)] = jnp.zeros((16,), jnp.float32)
            return carry
        lax.fori_loop(0, CHUNK, zero_body, 0)
        r0 = s * ROWS_PER_TILE
        for z in range(ROWS_PER_TILE // CHUNK):
            pltpu.sync_copy(buf0.at[pl.ds(0, CHUNK)],
                            acc.at[pl.ds(r0 + z * CHUNK, CHUNK)])
        plsc.subcore_barrier()

    def scale(buf, ebase):
        def scale_body(g, inner):
            vv = vals_v[pl.ds(ebase + g * 16, 16)]
            for i in range(16):
                v = vv[i]
                e = g * 16 + i
                for q in range(D // 16):
                    sl = pl.ds(q * 16, 16)
                    buf[e, sl] = buf[e, sl] * v
            return inner
        lax.fori_loop(0, CHUNK // 16, scale_body, 0)

    def process_round(nchunks):
        pltpu.async_copy(x_hbm.at[cols_v.at[0]], buf0, semg0)

        def pair_body(tt, carry):
            a0 = 2 * tt
            pltpu.make_async_copy(x_hbm.at[cols_v.at[a0]],
                                  buf0, semg0).wait()
            pltpu.async_copy(x_hbm.at[cols_v.at[a0 + 1]], buf1, semg1)
            scale(buf0, a0 * CHUNK)
            pltpu.sync_copy(buf0, acc.at[rows_v.at[a0]], add=True)
            pltpu.make_async_copy(x_hbm.at[cols_v.at[a0 + 1]],
                                  buf1, semg1).wait()

            @pl.when(tt < nchunks // 2 - 1)
            def _():
                pltpu.async_copy(x_hbm.at[cols_v.at[a0 + 2]], buf0, semg0)
            scale(buf1, (a0 + 1) * CHUNK)
            pltpu.sync_copy(buf1, acc.at[rows_v.at[a0 + 1]], add=True)
            return carry
        lax.fori_loop(0, nchunks // 2, pair_body, 0)

    def stage_and_run(rbase, nrows):
        pltpu.sync_copy(cols_hbm.at[pl.ds(rbase, nrows)],
                        cols_v.at[pl.ds(0, nrows)])
        pltpu.sync_copy(rows_hbm.at[pl.ds(rbase, nrows)],
                        rows_v.at[pl.ds(0, nrows)])
        pltpu.sync_copy(vals_hbm.at[pl.ds(rbase * CHUNK, nrows * CHUNK)],
                        vals_v.at[pl.ds(0, nrows * CHUNK)])
        process_round(nrows)

    with jax.named_scope("edge_loop"):
        @pl.when(wid < NTILES_A)
        def _():
            start = wid * ROWS_A
            stage_and_run(start, HROWS)
            stage_and_run(start + HROWS, ROWS_A - HROWS)

        @pl.when(wid >= NTILES_A)
        def _():
            start = NTILES_A * ROWS_A + (wid - NTILES_A) * ROWS_B
            stage_and_run(start, HROWS)
            stage_and_run(start + HROWS, ROWS_B - HROWS)

        @pl.when(wid == NW - 1)
        def _():
            pltpu.sync_copy(mcols_hbm, cols_v.at[pl.ds(0, MIXROWS)])
            pltpu.sync_copy(mrows_hbm, rows_v.at[pl.ds(0, MIXROWS)])
            pltpu.sync_copy(mvals_hbm,
                            vals_v.at[pl.ds(0, MIXROWS * CHUNK)])
            process_round(MIXROWS)

    with jax.named_scope("writeout"):
        plsc.subcore_barrier()
        pltpu.sync_copy(acc.at[pl.ds(r0, ROWS_PER_TILE)],
                        out_hbm.at[c].at[pl.ds(r0, ROWS_PER_TILE)])


_spmm_call = pl.kernel(
    _spmm_body,
    jax.ShapeDtypeStruct((NC, NACC, D), jnp.float32),
    mesh=_mesh,
    scratch_types=[
        pltpu.VMEM((HROWS, CHUNK), jnp.int32),
        pltpu.VMEM((HROWS, CHUNK), jnp.int32),
        pltpu.VMEM((HROWS * CHUNK,), jnp.float32),
        pltpu.VMEM((CHUNK, D), jnp.float32),
        pltpu.VMEM((CHUNK, D), jnp.float32),
        pltpu.VMEM_SHARED((NACC, D), jnp.float32),
        pltpu.SemaphoreType.DMA,
        pltpu.SemaphoreType.DMA,
    ],
)


BN = 2000


def _dense_body(x_ref, part_ref, w1t_ref, w2t_ref, bias_ref, a_ref, out_ref):
    p = part_ref[0] + part_ref[1]
    y = jnp.dot(x_ref[...], w1t_ref[...], preferred_element_type=jnp.float32)
    y = y + jnp.dot(p, w2t_ref[...], preferred_element_type=jnp.float32)
    y = y + bias_ref[...]
    a = a_ref[0]
    out_ref[...] = jnp.where(y >= 0.0, y, a * y)


def _dense(x, part, w1t, w2t, bias, a):
    return pl.pallas_call(
        _dense_body,
        grid=(N // BN,),
        in_specs=[
            pl.BlockSpec((BN, D), lambda i: (i, 0)),
            pl.BlockSpec((NC, BN, D), lambda i: (0, i, 0)),
            pl.BlockSpec((D, D), lambda i: (0, 0)),
            pl.BlockSpec((D, D), lambda i: (0, 0)),
            pl.BlockSpec((1, D), lambda i: (0, 0)),
            pl.BlockSpec(memory_space=pltpu.SMEM),
        ],
        out_specs=pl.BlockSpec((BN, D), lambda i: (i, 0)),
        out_shape=jax.ShapeDtypeStruct((N, D), jnp.float32),
    )(x, part, w1t, w2t, bias.reshape(1, D), a.reshape(1))


def kernel(X, edge_index, edge_vals, W1_0, b1_0, W2_0, b2_0, a_0,
           W1_1, b1_1, W2_1, b2_1, a_1):
    nmix = MIXROWS * CHUNK // 2
    ebase = E - nmix
    rows_r = edge_index[0].reshape(NROWS_E, CHUNK)
    cols_r = edge_index[1].reshape(NROWS_E, CHUNK)
    fill = jnp.arange(nmix, dtype=jnp.int32)
    mrows = jnp.concatenate(
        [edge_index[0][ebase:], N + fill % (NACC - N)]).reshape(MIXROWS, CHUNK)
    mcols = jnp.concatenate(
        [edge_index[1][ebase:], fill % N]).reshape(MIXROWS, CHUNK)
    mvals = jnp.concatenate(
        [edge_vals[ebase:], jnp.zeros((nmix,), jnp.float32)])

    part = _spmm_call(X, rows_r, cols_r, edge_vals, mrows, mcols, mvals)
    t1 = _dense(X, part, W1_0.T, W2_0.T, b1_0 + b2_0, a_0)
    part = _spmm_call(t1, rows_r, cols_r, edge_vals, mrows, mcols, mvals)
    t2 = _dense(t1, part, W1_1.T, W2_1.T, b1_1 + b2_1, a_1)
    return jnp.expand_dims(t2, 0)


# speedup vs baseline: 1.0138x; 1.0138x over previous
import jax
import jax.numpy as jnp
from jax import lax
from jax.experimental import pallas as pl
from jax.experimental.pallas import tpu as pltpu
from jax.experimental.pallas import tpu_sc as plsc

N = 10000
D = 128
E = 320000

NC = 2
NS = 16
NW = NC * NS
CHUNK = 128
EPT = 10240
HALVES = 2
HEDGES = EPT // HALVES
HROWS = HEDGES // CHUNK
CPH = HEDGES // CHUNK
EPAD = NW * EPT
NACC = 10240
ROWS_PER_TILE = NACC // NS

_mesh = plsc.VectorSubcoreMesh(core_axis_name="c", subcore_axis_name="s")


def _spmm_body(x_hbm, rows_hbm, cols_hbm, vals_hbm, out_hbm,
               cols_v, rows_v, vals_v, buf0, buf1, acc, semg0, semg1):
    c = lax.axis_index("c")
    s = lax.axis_index("s")
    wid = s * NC + c

    with jax.named_scope("acc_zero"):
        def zero_body(e, carry):
            for q in range(D // 16):
                buf0[e, pl.ds(q * 16, 16)] = jnp.zeros((16,), jnp.float32)
            return carry
        lax.fori_loop(0, CHUNK, zero_body, 0)
        r0 = s * ROWS_PER_TILE
        for z in range(ROWS_PER_TILE // CHUNK):
            pltpu.sync_copy(buf0.at[pl.ds(0, CHUNK)],
                            acc.at[pl.ds(r0 + z * CHUNK, CHUNK)])
        plsc.subcore_barrier()

    nrows_idx = EPT // CHUNK

    def scale(buf, ebase):
        def scale_body(g, inner):
            vv = vals_v[pl.ds(ebase + g * 16, 16)]
            for i in range(16):
                v = vv[i]
                e = g * 16 + i
                for q in range(D // 16):
                    sl = pl.ds(q * 16, 16)
                    buf[e, sl] = buf[e, sl] * v
            return inner
        lax.fori_loop(0, CHUNK // 16, scale_body, 0)

    with jax.named_scope("edge_loop"):
        for h in range(HALVES):
            rbase = wid * nrows_idx + h * HROWS
            pltpu.sync_copy(cols_hbm.at[pl.ds(rbase, HROWS)], cols_v)
            pltpu.sync_copy(rows_hbm.at[pl.ds(rbase, HROWS)], rows_v)
            pltpu.sync_copy(
                vals_hbm.at[pl.ds(wid * EPT + h * HEDGES, HEDGES)], vals_v)

            pltpu.async_copy(x_hbm.at[cols_v.at[0]], buf0, semg0)

            def pair_body(tt, carry):
                a0 = 2 * tt
                pltpu.make_async_copy(x_hbm.at[cols_v.at[a0]],
                                      buf0, semg0).wait()
                pltpu.async_copy(x_hbm.at[cols_v.at[a0 + 1]], buf1, semg1)
                scale(buf0, a0 * CHUNK)
                pltpu.sync_copy(buf0, acc.at[rows_v.at[a0]], add=True)
                pltpu.make_async_copy(x_hbm.at[cols_v.at[a0 + 1]],
                                      buf1, semg1).wait()

                @pl.when(tt < CPH // 2 - 1)
                def _():
                    pltpu.async_copy(x_hbm.at[cols_v.at[a0 + 2]], buf0, semg0)
                scale(buf1, (a0 + 1) * CHUNK)
                pltpu.sync_copy(buf1, acc.at[rows_v.at[a0 + 1]], add=True)
                return carry
            lax.fori_loop(0, CPH // 2, pair_body, 0)

    with jax.named_scope("writeout"):
        plsc.subcore_barrier()
        pltpu.sync_copy(acc.at[pl.ds(r0, ROWS_PER_TILE)],
                        out_hbm.at[c].at[pl.ds(r0, ROWS_PER_TILE)])


_spmm_call = pl.kernel(
    _spmm_body,
    jax.ShapeDtypeStruct((NC, NACC, D), jnp.float32),
    mesh=_mesh,
    scratch_types=[
        pltpu.VMEM((HROWS, CHUNK), jnp.int32),
        pltpu.VMEM((HROWS, CHUNK), jnp.int32),
        pltpu.VMEM((HEDGES,), jnp.float32),
        pltpu.VMEM((CHUNK, D), jnp.float32),
        pltpu.VMEM((CHUNK, D), jnp.float32),
        pltpu.VMEM_SHARED((NACC, D), jnp.float32),
        pltpu.SemaphoreType.DMA,
        pltpu.SemaphoreType.DMA,
    ],
)


BN = 2000


def _dense_body(x_ref, part_ref, w1t_ref, w2t_ref, bias_ref, a_ref, out_ref):
    p = part_ref[0] + part_ref[1]
    y = jnp.dot(x_ref[...], w1t_ref[...], preferred_element_type=jnp.float32)
    y = y + jnp.dot(p, w2t_ref[...], preferred_element_type=jnp.float32)
    y = y + bias_ref[...]
    a = a_ref[0]
    out_ref[...] = jnp.where(y >= 0.0, y, a * y)


def _dense(x, part, w1t, w2t, bias, a):
    return pl.pallas_call(
        _dense_body,
        grid=(N // BN,),
        in_specs=[
            pl.BlockSpec((BN, D), lambda i: (i, 0)),
            pl.BlockSpec((NC, BN, D), lambda i: (0, i, 0)),
            pl.BlockSpec((D, D), lambda i: (0, 0)),
            pl.BlockSpec((D, D), lambda i: (0, 0)),
            pl.BlockSpec((1, D), lambda i: (0, 0)),
            pl.BlockSpec(memory_space=pltpu.SMEM),
        ],
        out_specs=pl.BlockSpec((BN, D), lambda i: (i, 0)),
        out_shape=jax.ShapeDtypeStruct((N, D), jnp.float32),
    )(x, part, w1t, w2t, bias.reshape(1, D), a.reshape(1))


def kernel(X, edge_index, edge_vals, W1_0, b1_0, W2_0, b2_0, a_0,
           W1_1, b1_1, W2_1, b2_1, a_1):
    pad = EPAD - E
    pad_rows = N + (jnp.arange(pad, dtype=jnp.int32) % (NACC - N))
    pad_cols = jnp.arange(pad, dtype=jnp.int32) % N
    rows2 = jnp.concatenate(
        [edge_index[0], pad_rows]).reshape(EPAD // CHUNK, CHUNK)
    cols2 = jnp.concatenate(
        [edge_index[1], pad_cols]).reshape(EPAD // CHUNK, CHUNK)
    vals1 = jnp.concatenate([edge_vals, jnp.zeros((pad,), jnp.float32)])

    part = _spmm_call(X, rows2, cols2, vals1)
    t1 = _dense(X, part, W1_0.T, W2_0.T, b1_0 + b2_0, a_0)
    part = _spmm_call(t1, rows2, cols2, vals1)
    t2 = _dense(t1, part, W1_1.T, W2_1.T, b1_1 + b2_1, a_1)
    return jnp.expand_dims(t2, 0)
